# Initial kernel scaffold; baseline (speedup 1.0000x reference)
import jax
import jax.numpy as jnp
from jax.experimental import pallas as pl


def kernel(tree):
    raise NotImplementedError("write your pallas kernel here")


# speedup vs baseline: 287.5227x; 287.5227x over previous
import functools

import jax
import jax.numpy as jnp
from jax import lax
from jax.experimental import pallas as pl
from jax.experimental.pallas import tpu as pltpu
from jax.experimental.pallas import tpu_sc as plsc

_CLASS_MAP = (19, 19, 19, 19, 19, 19, 19, 0, 1, 19, 19, 2, 3, 4, 19, 19, 19, 5,
              19, 6, 7, 8, 9, 10, 11, 12, 13, 14, 15, 19, 19, 16, 17, 18, 19)
_TABLE_PAD = 64

_N = 16 * 512 * 512
_NC, _NS, _L = 2, 16, 16
_NW = _NC * _NS
_PER_W = _N // _NW
_BLK = 32768
_NBLK = _PER_W // _BLK
_NVEC = _BLK // _L

_mesh = plsc.VectorSubcoreMesh(core_axis_name="c", subcore_axis_name="s",
                               num_cores=_NC, num_subcores=_NS)


@functools.partial(
    pl.kernel,
    out_type=jax.ShapeDtypeStruct((_N,), jnp.int32),
    mesh=_mesh,
    compiler_params=pltpu.CompilerParams(needs_layout_passes=False),
    scratch_types=[
        pltpu.VMEM((_TABLE_PAD,), jnp.int32),
        pltpu.VMEM((_BLK,), jnp.int32),
        pltpu.SemaphoreType.DMA,
    ],
)
def _remap(tree_hbm, table_hbm, out_hbm, table_v, buf, sem):
  wid = lax.axis_index("s") * _NC + lax.axis_index("c")
  base = wid * _PER_W

  pltpu.sync_copy(table_hbm, table_v)

  def vec_body(i, carry):
    idx = buf[pl.ds(i * _L, _L)]
    buf[pl.ds(i * _L, _L)] = plsc.load_gather(table_v, [idx])
    return carry

  for b in range(_NBLK):
    off = base + b * _BLK
    pltpu.sync_copy(tree_hbm.at[pl.ds(off, _BLK)], buf)
    lax.fori_loop(0, _NVEC, vec_body, 0)
    pltpu.sync_copy(buf, out_hbm.at[pl.ds(off, _BLK)])


def kernel(tree):
  table = jnp.zeros((_TABLE_PAD,), jnp.int32).at[:35].set(
      jnp.asarray(_CLASS_MAP, jnp.int32))
  flat = tree.reshape(_N).astype(jnp.int32)
  out = _remap(flat, table)
  return out.reshape(tree.shape).astype(tree.dtype)


# speedup vs baseline: 537.2217x; 1.8684x over previous
import functools

import jax
import jax.numpy as jnp
from jax import lax
from jax.experimental import pallas as pl
from jax.experimental.pallas import tpu as pltpu
from jax.experimental.pallas import tpu_sc as plsc

_CLASS_MAP = (19, 19, 19, 19, 19, 19, 19, 0, 1, 19, 19, 2, 3, 4, 19, 19, 19, 5,
              19, 6, 7, 8, 9, 10, 11, 12, 13, 14, 15, 19, 19, 16, 17, 18, 19)
_TABLE_PAD = 64

_N = 16 * 512 * 512
_NC, _NS, _L = 2, 16, 16
_NW = _NC * _NS
_PER_W = _N // _NW
_BLK = 32768
_NBLK = _PER_W // _BLK
_NVEC = _BLK // _L

_mesh = plsc.VectorSubcoreMesh(core_axis_name="c", subcore_axis_name="s",
                               num_cores=_NC, num_subcores=_NS)


@functools.partial(
    pl.kernel,
    out_type=jax.ShapeDtypeStruct((_N,), jnp.int32),
    mesh=_mesh,
    compiler_params=pltpu.CompilerParams(needs_layout_passes=False),
    scratch_types=[
        pltpu.VMEM((_TABLE_PAD,), jnp.int32),
        pltpu.VMEM((_BLK,), jnp.int32),
        pltpu.SemaphoreType.DMA,
    ],
)
def _remap(tree_hbm, table_hbm, out_hbm, table_v, buf, sem):
  wid = lax.axis_index("s") * _NC + lax.axis_index("c")
  base = wid * _PER_W

  pltpu.sync_copy(table_hbm, table_v)

  for b in range(_NBLK):
    off = base + b * _BLK
    pltpu.sync_copy(tree_hbm.at[pl.ds(off, _BLK)], buf)

    @plsc.parallel_loop(0, _BLK, _L, unroll=8)
    def vec_body(i):
      idx = buf[pl.ds(i, _L)]
      buf[pl.ds(i, _L)] = plsc.load_gather(table_v, [idx])

    pltpu.sync_copy(buf, out_hbm.at[pl.ds(off, _BLK)])


def kernel(tree):
  table = jnp.zeros((_TABLE_PAD,), jnp.int32).at[:35].set(
      jnp.asarray(_CLASS_MAP, jnp.int32))
  flat = tree.reshape(_N).astype(jnp.int32)
  out = _remap(flat, table)
  return out.reshape(tree.shape).astype(tree.dtype)
